# baseline (speedup 1.0000x reference)
import jax
import jax.numpy as jnp
from jax import lax
from jax.experimental import pallas as pl
from jax.experimental.pallas import tpu as pltpu
from jax.experimental.pallas import tpu_sc as plsc
import numpy as np

BATCH = 64
K_SC = 32
NW = 32

CROWS = 16
NCHUNK = 128 // CROWS
GUNROLL = 8

THRESH = float(2.0 * np.sin(np.deg2rad(2.0) / 2.0))
THRESH2 = THRESH ** 2
ALPHA = 1.0
BETA = 1.0


def _sc_partials_kernel(z_hbm, r_hbm, part_hbm, r_v, bufs, acc_v, sems):
    wid = lax.axis_index("s") * 2 + lax.axis_index("c")
    pltpu.sync_copy(r_hbm, r_v)
    b = wid

    def start(c, par):
        return [
            pltpu.async_copy(
                z_hbm.at[b, comp, pl.ds(c * CROWS, CROWS), :],
                bufs[par][comp], sems[par])
            for comp in range(3)
        ]

    rvec = plsc.load_gather(r_v, [jnp.full((16,), b, jnp.int32)])
    bvec = -0.5 / (rvec * rvec)

    NACC = 4
    accs = tuple(jnp.zeros((16,), jnp.float32) for _ in range(NACC))
    descs = start(0, 0)
    for c in range(NCHUNK):
        par = c % 2
        if c + 1 < NCHUNK:
            nxt = start(c + 1, 1 - par)
        for d in descs:
            d.wait()
        bx, by, bz = bufs[par]

        def row(r, accs):
            def step(g, accs):
                accs = list(accs)
                base = g * (16 * GUNROLL)
                for k in range(GUNROLL):
                    col = pl.ds(base + k * 16, 16)
                    x = bx[r, col]
                    y = by[r, col]
                    z = bz[r, col]
                    z2 = x * x + y * y + z * z
                    e = jnp.exp(bvec * z2)
                    accs[k % NACC] = accs[k % NACC] + jnp.where(
                        z2 < THRESH2, e, 0.0)
                return tuple(accs)
            return lax.fori_loop(0, 1024 // (16 * GUNROLL), step, accs)

        accs = lax.fori_loop(0, CROWS, row, accs)
        if c + 1 < NCHUNK:
            descs = nxt

    acc = accs[0]
    for a in accs[1:]:
        acc = acc + a
    acc_v[...] = acc
    pltpu.sync_copy(acc_v, part_hbm.at[b])


@jax.jit
def _sc_partials(zt, r):
    kfn = pl.kernel(
        _sc_partials_kernel,
        out_type=jax.ShapeDtypeStruct((K_SC, 16), jnp.float32),
        mesh=plsc.VectorSubcoreMesh(core_axis_name="c", subcore_axis_name="s"),
        scratch_types=[
            pltpu.VMEM((BATCH,), jnp.float32),
            [[pltpu.VMEM((CROWS, 1024), jnp.float32) for _ in range(3)]
             for _ in range(2)],
            pltpu.VMEM((16,), jnp.float32),
            [pltpu.SemaphoreType.DMA, pltpu.SemaphoreType.DMA],
        ],
        compiler_params=pltpu.CompilerParams(
            needs_layout_passes=False, use_tc_tiling_on_sc=True),
    )
    return kfn(zt, r)


def _tc_reduce_kernel(z_ref, r_ref, out_ref):
    x = z_ref[0, 0]
    y = z_ref[0, 1]
    z = z_ref[0, 2]
    z2 = x * x + y * y + z * z
    rv = r_ref[pl.program_id(0) + K_SC]
    bv = -0.5 / (rv * rv)
    e = jnp.exp(bv * z2)
    s = jnp.sum(jnp.where(z2 < THRESH2, e, 0.0))
    out_ref[pl.program_id(0)] = s


def _tc_reduce(zt, r):
    n = BATCH - K_SC
    return pl.pallas_call(
        _tc_reduce_kernel,
        grid=(n,),
        in_specs=[
            pl.BlockSpec((1, 3, 128, 1024), lambda i: (i + K_SC, 0, 0, 0)),
            pl.BlockSpec(memory_space=pltpu.SMEM),
        ],
        out_specs=pl.BlockSpec(memory_space=pltpu.SMEM),
        out_shape=jax.ShapeDtypeStruct((n,), jnp.float32),
    )(zt, r)


def _tc_epilogue_kernel(part_ref, tc_ref, r_ref, n_ref,
                        raw_ref, mu_ref, s2_ref, obj_ref):
    sc_raw = jnp.sum(part_ref[...], axis=1)
    raw = jnp.concatenate([sc_raw, tc_ref[...]], axis=0)
    r = r_ref[...]
    n = n_ref[0]
    a = 1.0 / (r * r)
    lam = (0.5 * THRESH2) * a
    mu_per = (1.0 - jnp.exp(-lam)) / lam
    e2 = (1.0 - jnp.exp(-2.0 * lam)) / (2.0 * lam)
    sig2_per = e2 - mu_per * mu_per
    mu = n * mu_per
    sigma2 = n * sig2_per
    sigma = jnp.sqrt(sigma2)
    raw_ref[...] = raw
    mu_ref[...] = mu
    s2_ref[...] = sigma2
    obj_ref[...] = raw - ALPHA * mu - BETA + sigma


def _tc_epilogue(part, tc_raw, r, num_obs):
    out_shape = jax.ShapeDtypeStruct((BATCH,), jnp.float32)
    return pl.pallas_call(
        _tc_epilogue_kernel,
        out_shape=(out_shape, out_shape, out_shape, out_shape),
        in_specs=[
            pl.BlockSpec(memory_space=pltpu.VMEM),
            pl.BlockSpec(memory_space=pltpu.VMEM),
            pl.BlockSpec(memory_space=pltpu.VMEM),
            pl.BlockSpec(memory_space=pltpu.SMEM),
        ],
        out_specs=(
            pl.BlockSpec(memory_space=pltpu.VMEM),
            pl.BlockSpec(memory_space=pltpu.VMEM),
            pl.BlockSpec(memory_space=pltpu.VMEM),
            pl.BlockSpec(memory_space=pltpu.VMEM),
        ),
    )(part, tc_raw, r, num_obs)


def kernel(z, R, num_obs):
    zt = jnp.transpose(z, (0, 3, 1, 2))
    part = _sc_partials(zt, R)
    tc_raw = _tc_reduce(zt, R)
    n1 = jnp.reshape(jnp.asarray(num_obs, jnp.float32), (1,))
    raw, mu, sigma2, obj = _tc_epilogue(part, tc_raw, R, n1)
    return (raw, mu, sigma2, obj)
